# baseline (speedup 1.0000x reference)
import functools

import jax
import jax.numpy as jnp
from jax import lax
from jax.experimental import pallas as pl
from jax.experimental.pallas import tpu as pltpu
from jax.experimental.pallas import tpu_sc as plsc

_N_REP = 16


def _build_sc_repeat(seq, d):
    out_w = d * _N_REP
    info = plsc.get_sparse_core_info()
    nc, ns, lanes = info.num_cores, info.num_subcores, info.num_lanes
    nw = nc * ns
    rows_w = seq // nw
    chunk = 32
    n_chunks = rows_w // chunk
    n_vec = d // lanes

    mesh = plsc.VectorSubcoreMesh(core_axis_name="c", subcore_axis_name="s")

    @functools.partial(
        pl.kernel,
        mesh=mesh,
        out_type=jax.ShapeDtypeStruct((seq * out_w,), jnp.float32),
        scratch_types=[
            pltpu.VMEM((rows_w * d,), jnp.float32),
            pltpu.VMEM((chunk * out_w,), jnp.float32),
            pltpu.VMEM((chunk * out_w,), jnp.float32),
            pltpu.SemaphoreType.DMA,
            pltpu.SemaphoreType.DMA,
        ],
    )
    def k(table_hbm, out_hbm, tbl_v, ob0, ob1, sem0, sem1):
        wid = lax.axis_index("s") * nc + lax.axis_index("c")
        row0 = wid * rows_w
        pltpu.sync_copy(table_hbm.at[pl.ds(row0 * d, rows_w * d)], tbl_v)

        obufs = (ob0, ob1)
        sems = (sem0, sem1)
        pending = [None, None]

        dn = lax.GatherDimensionNumbers(
            offset_dims=(), collapsed_slice_dims=(0,), start_index_map=(0,))

        for ck in range(n_chunks):
            b = ck % 2
            if pending[b] is not None:
                pending[b].wait()
                pending[b] = None
            ob = obufs[b]

            @plsc.parallel_loop(0, chunk, 1, unroll=8)
            def row_body(rr, ck=ck, ob=ob):
                for v in range(n_vec):
                    x = tbl_v[pl.ds((ck * chunk + rr) * d + v * lanes, lanes)]
                    for l in range(lanes):
                        idx = jnp.full((lanes, 1), l, dtype=jnp.int32)
                        y = lax.gather(
                            x, idx, dn, slice_sizes=(1,),
                            mode=lax.GatherScatterMode.PROMISE_IN_BOUNDS)
                        ob[pl.ds(rr * out_w + (v * lanes + l) * _N_REP,
                                 _N_REP)] = y
            pending[b] = pltpu.async_copy(
                ob,
                out_hbm.at[pl.ds((row0 + ck * chunk) * out_w, chunk * out_w)],
                sems[b])

        for b in range(2):
            if pending[b] is not None:
                pending[b].wait()

    return k


def kernel(inputs, pos_table):
    seq = inputs.shape[-2]
    d = pos_table.shape[-1]
    flat = _build_sc_repeat(seq, d)(pos_table[:seq].reshape(seq * d))
    return flat.reshape(seq, d * _N_REP)


# speedup vs baseline: 1.0631x; 1.0631x over previous
import functools

import jax
import jax.numpy as jnp
from jax import lax
from jax.experimental import pallas as pl
from jax.experimental.pallas import tpu as pltpu
from jax.experimental.pallas import tpu_sc as plsc

_N_REP = 16


def _build_sc_repeat(seq, d):
    out_w = d * _N_REP
    info = plsc.get_sparse_core_info()
    nc, ns, lanes = info.num_cores, info.num_subcores, info.num_lanes
    nw = nc * ns
    rows_w = seq // nw
    chunk = 32
    n_chunks = rows_w // chunk
    n_vec = d // lanes

    mesh = plsc.VectorSubcoreMesh(core_axis_name="c", subcore_axis_name="s")

    @functools.partial(
        pl.kernel,
        mesh=mesh,
        out_type=jax.ShapeDtypeStruct((seq * out_w,), jnp.float32),
        scratch_types=[
            pltpu.VMEM((rows_w * d,), jnp.float32),
            pltpu.VMEM((chunk * out_w,), jnp.float32),
            pltpu.VMEM((chunk * out_w,), jnp.float32),
            pltpu.SemaphoreType.DMA,
            pltpu.SemaphoreType.DMA,
        ],
    )
    def k(table_hbm, out_hbm, tbl_v, ob0, ob1, sem0, sem1):
        wid = lax.axis_index("s") * nc + lax.axis_index("c")
        row0 = wid * rows_w
        pltpu.sync_copy(table_hbm.at[pl.ds(row0 * d, rows_w * d)], tbl_v)

        obufs = (ob0, ob1)
        sems = (sem0, sem1)
        pending = [None, None]

        dn = lax.GatherDimensionNumbers(
            offset_dims=(), collapsed_slice_dims=(0,), start_index_map=(0,))

        for ck in range(n_chunks):
            b = ck % 2
            if pending[b] is not None:
                pending[b].wait()
                pending[b] = None
            ob = obufs[b]

            @plsc.parallel_loop(0, chunk, 1, unroll=2)
            def row_body(rr, ck=ck, ob=ob):
                for v in range(n_vec):
                    x = tbl_v[pl.ds((ck * chunk + rr) * d + v * lanes, lanes)]
                    for l in range(lanes):
                        idx = jnp.full((lanes, 1), l, dtype=jnp.int32)
                        y = lax.gather(
                            x, idx, dn, slice_sizes=(1,),
                            mode=lax.GatherScatterMode.PROMISE_IN_BOUNDS)
                        ob[pl.ds(rr * out_w + (v * lanes + l) * _N_REP,
                                 _N_REP)] = y
            pending[b] = pltpu.async_copy(
                ob,
                out_hbm.at[pl.ds((row0 + ck * chunk) * out_w, chunk * out_w)],
                sems[b])

        for b in range(2):
            if pending[b] is not None:
                pending[b].wait()

    return k


def kernel(inputs, pos_table):
    seq = inputs.shape[-2]
    d = pos_table.shape[-1]
    flat = _build_sc_repeat(seq, d)(pos_table[:seq].reshape(seq * d))
    return flat.reshape(seq, d * _N_REP)


# speedup vs baseline: 1.0822x; 1.0179x over previous
import functools

import jax
import jax.numpy as jnp
from jax import lax
from jax.experimental import pallas as pl
from jax.experimental.pallas import tpu as pltpu
from jax.experimental.pallas import tpu_sc as plsc

_N_REP = 16


def _build_sc_repeat(seq, d):
    out_w = d * _N_REP
    info = plsc.get_sparse_core_info()
    nc, ns, lanes = info.num_cores, info.num_subcores, info.num_lanes
    nw = nc * ns
    rows_w = seq // nw
    chunk = 32
    n_chunks = rows_w // chunk
    n_vec = d // lanes

    mesh = plsc.VectorSubcoreMesh(core_axis_name="c", subcore_axis_name="s")

    @functools.partial(
        pl.kernel,
        mesh=mesh,
        out_type=jax.ShapeDtypeStruct((seq * out_w,), jnp.float32),
        scratch_types=[
            pltpu.VMEM((rows_w * d,), jnp.float32),
            pltpu.VMEM((chunk * out_w,), jnp.float32),
            pltpu.VMEM((chunk * out_w,), jnp.float32),
            pltpu.SemaphoreType.DMA,
            pltpu.SemaphoreType.DMA,
        ],
    )
    def k(table_hbm, out_hbm, tbl_v, ob0, ob1, sem0, sem1):
        wid = lax.axis_index("s") * nc + lax.axis_index("c")
        row0 = wid * rows_w
        pltpu.sync_copy(table_hbm.at[pl.ds(row0 * d, rows_w * d)], tbl_v)

        obufs = (ob0, ob1)
        sems = (sem0, sem1)
        pending = [None, None]

        dn = lax.GatherDimensionNumbers(
            offset_dims=(), collapsed_slice_dims=(0,), start_index_map=(0,))

        for ck in range(n_chunks):
            b = ck % 2
            if pending[b] is not None:
                pending[b].wait()
                pending[b] = None
            ob = obufs[b]

            @plsc.parallel_loop(0, chunk, 1)
            def row_body(rr, ck=ck, ob=ob):
                for v in range(n_vec):
                    x = tbl_v[pl.ds((ck * chunk + rr) * d + v * lanes, lanes)]
                    for l in range(lanes):
                        idx = jnp.full((lanes, 1), l, dtype=jnp.int32)
                        y = lax.gather(
                            x, idx, dn, slice_sizes=(1,),
                            mode=lax.GatherScatterMode.PROMISE_IN_BOUNDS)
                        ob[pl.ds(rr * out_w + (v * lanes + l) * _N_REP,
                                 _N_REP)] = y
            pending[b] = pltpu.async_copy(
                ob,
                out_hbm.at[pl.ds((row0 + ck * chunk) * out_w, chunk * out_w)],
                sems[b])

        for b in range(2):
            if pending[b] is not None:
                pending[b].wait()

    return k


def kernel(inputs, pos_table):
    seq = inputs.shape[-2]
    d = pos_table.shape[-1]
    flat = _build_sc_repeat(seq, d)(pos_table[:seq].reshape(seq * d))
    return flat.reshape(seq, d * _N_REP)
